# Initial kernel scaffold; baseline (speedup 1.0000x reference)
import jax
import jax.numpy as jnp
from jax.experimental import pallas as pl


def kernel(x_0, timesteps, noise, alphas_cumprod):
    raise NotImplementedError("write your pallas kernel here")


# speedup vs baseline: 1.7772x; 1.7772x over previous
import functools

import jax
import jax.numpy as jnp
from jax.experimental import pallas as pl
from jax.experimental.pallas import tpu as pltpu

_B = 16384
_D = 1024
_NT = 1000
_TPAD = 1024
_ROWS = 256
_NB = _B // _ROWS


def _block_kernel(ts_ref, tbl_ref, x0_ref, nz_ref, out_ref):
    ts = ts_ref[...]
    tbl = tbl_ref[...]
    k = jax.lax.broadcasted_iota(jnp.int32, (_ROWS, _TPAD), 1)
    ac = jnp.sum(jnp.where(k == ts, tbl, 0.0), axis=1, keepdims=True)
    sa = jnp.sqrt(ac)
    sb = jnp.sqrt(1.0 - ac)
    out_ref[...] = sa * x0_ref[...] + sb * nz_ref[...]


@jax.jit
def kernel(x_0, timesteps, noise, alphas_cumprod):
    tbl = jnp.pad(alphas_cumprod, (0, _TPAD - _NT)).reshape(1, _TPAD)
    return pl.pallas_call(
        _block_kernel,
        grid=(_NB,),
        in_specs=[
            pl.BlockSpec((_ROWS, 1), lambda i: (i, 0)),
            pl.BlockSpec((1, _TPAD), lambda i: (0, 0)),
            pl.BlockSpec((_ROWS, _D), lambda i: (i, 0)),
            pl.BlockSpec((_ROWS, _D), lambda i: (i, 0)),
        ],
        out_specs=pl.BlockSpec((_ROWS, _D), lambda i: (i, 0)),
        out_shape=jax.ShapeDtypeStruct((_B, _D), jnp.float32),
        compiler_params=pltpu.CompilerParams(
            dimension_semantics=("arbitrary",),
        ),
    )(timesteps, tbl, x_0, noise)
